# baseline (device time: 363756 ns/iter reference)
import jax
import jax.numpy as jnp
from jax import lax
from jax.experimental import pallas as pl
from jax.experimental.pallas import tpu as pltpu

T = 1024
D = 2048
VH = 16384
NCHUNK = 32
CW = VH // NCHUNK
NQ = 4
NM = NCHUNK // NQ
GW = NQ * CW
F32 = jnp.float32


def _fused(x, W):
    def body(x_ref, w_ref, out_ref, recv_ref, emine_ref, comm_ref, estage,
             s_acc, s_other, out_stage, in_stage,
             z_send, z_recv, xd_send, xd_recv, yd_send, yd_recv,
             yt_send, yt_recv, xt_send, xt_recv,
             stat_send, stat_recv, in_copy, out_copy, ecopy):
        j = pl.program_id(0)
        my_x = lax.axis_index("x")
        my_y = lax.axis_index("y")
        my_z = lax.axis_index("z")
        zpeer = (my_x, my_y, 1 - my_z)
        xpeer = (1 - my_x, my_y, my_z)
        ypeer = (my_x, 1 - my_y, my_z)

        q_me = my_x + 2 * my_y
        q_xp = (1 - my_x) + 2 * my_y
        q_yp = my_x + 2 * (1 - my_y)
        q_d = (1 - my_x) + 2 * (1 - my_y)

        def rdma(src, h, send_sem, recv_sem, peer):
            return pltpu.make_async_remote_copy(
                src_ref=src,
                dst_ref=recv_ref.at[h],
                send_sem=send_sem,
                recv_sem=recv_sem,
                device_id=peer,
                device_id_type=pl.DeviceIdType.MESH,
            )

        def z_rdma(m):
            return rdma(comm_ref.at[m], NQ * m + q_me,
                        z_send.at[m], z_recv.at[m], zpeer)

        def xd_out(m):
            h = NQ * m + q_me
            return rdma(recv_ref.at[h], h, xd_send.at[m], xd_recv.at[m],
                        xpeer)

        def yd_out(m):
            h = NQ * m + q_me
            return rdma(recv_ref.at[h], h, yd_send.at[m], yd_recv.at[m],
                        ypeer)

        def xd_in(m):
            h = NQ * m + q_xp
            return rdma(recv_ref.at[h], h, xd_send.at[m], xd_recv.at[m],
                        xpeer)

        def yd_in(m):
            h = NQ * m + q_yp
            return rdma(recv_ref.at[h], h, yd_send.at[m], yd_recv.at[m],
                        ypeer)

        def yt_out(m):
            h = NQ * m + q_xp
            return rdma(recv_ref.at[h], h, yt_send.at[m], yt_recv.at[m],
                        ypeer)

        def yt_in(m):
            h = NQ * m + q_d
            return rdma(recv_ref.at[h], h, yt_send.at[m], yt_recv.at[m],
                        ypeer)

        def xt_out(m):
            h = NQ * m + q_yp
            return rdma(recv_ref.at[h], h, xt_send.at[m - 4],
                        xt_recv.at[m - 4], xpeer)

        def xt_in(m):
            h = NQ * m + q_d
            return rdma(recv_ref.at[h], h, xt_send.at[m - 4],
                        xt_recv.at[m - 4], xpeer)

        def stat_rdma():
            return pltpu.make_async_remote_copy(
                src_ref=s_acc,
                dst_ref=s_other,
                send_sem=stat_send,
                recv_sem=stat_recv,
                device_id=zpeer,
                device_id_type=pl.DeviceIdType.MESH,
            )

        @pl.when(j == 0)
        def _():
            barrier = pltpu.get_barrier_semaphore()
            for peer in (zpeer, xpeer, ypeer):
                pl.semaphore_signal(
                    barrier, inc=1, device_id=peer,
                    device_id_type=pl.DeviceIdType.MESH,
                )
            pl.semaphore_wait(barrier, 3)

        xb = x_ref[...].astype(jnp.bfloat16)
        wb = w_ref[...].astype(jnp.bfloat16)
        acc = jnp.dot(xb, wb, preferred_element_type=F32)
        eacc = jnp.exp(acc)
        eb = eacc.astype(jnp.bfloat16)
        ex_sum = jnp.sum(eacc, axis=1, keepdims=True)

        def emine_copy(h, slot):
            return pltpu.make_async_copy(
                estage.at[slot], emine_ref.at[h], ecopy.at[slot]
            )

        @pl.when(j >= 2)
        def _():
            emine_copy(j - 2, j % 2).wait()

        estage[j % 2] = eb
        emine_copy(j, j % 2).start()

        @pl.when(j == 0)
        def _():
            s_acc[...] = ex_sum

        @pl.when(j > 0)
        def _():
            s_acc[...] = s_acc[...] + ex_sum

        @pl.when(j % NQ == q_me)
        def _():
            m = j // NQ
            comm_ref[m] = eb
            z_rdma(m).start()

        @pl.when((j >= 4) & (j % NQ == 0))
        def _():
            m = j // NQ - 1
            z_rdma(m).wait_recv()
            xd_out(m).start()
            yd_out(m).start()

        @pl.when((j >= 20) & (j <= 29) & ((j - 20) % 3 == 0))
        def _():
            m = (j - 20) // 3
            xd_in(m).wait_recv()
            yt_out(m).start()

        @pl.when(j == NCHUNK - 1)
        def _():
            stat_rdma().start()
            for slot in range(2):
                emine_copy(NCHUNK - 2 + slot, slot).wait()

            z_rdma(NM - 1).wait_recv()
            xd_out(NM - 1).start()
            yd_out(NM - 1).start()

            for m in range(4, NM):
                yd_in(m).wait_recv()
                xt_out(m).start()

            stat_rdma().wait_recv()
            inv = 1.0 / (s_acc[...] + s_other[...])

            items = [
                (emine_ref.at[pl.ds(NQ * g, NQ)], [], my_z * VH + g * GW)
                for g in range(NM)
            ] + [
                (
                    recv_ref.at[pl.ds(NQ * g, NQ)],
                    ([yd_in(g), yt_in(g)] if g < 4
                     else [xd_in(g), xt_in(g)]),
                    (1 - my_z) * VH + g * GW,
                )
                for g in range(NM)
            ]
            n = len(items)

            def copy_in(i):
                src, _, _ = items[i]
                return pltpu.make_async_copy(
                    src, in_stage.at[i % 2], in_copy.at[i % 2]
                )

            def start_in(i):
                _, waits, _ = items[i]
                for wd in waits:
                    wd.wait_recv()
                copy_in(i).start()

            start_in(0)
            for i in range(n):
                _, _, out_col = items[i]
                if i + 1 < n:
                    start_in(i + 1)
                copy_in(i).wait()
                slot = i % 2
                if i >= 2:
                    pltpu.make_async_copy(
                        out_stage.at[slot],
                        out_ref.at[:, pl.ds(out_col, GW)],
                        out_copy.at[slot],
                    ).wait()
                for q in range(NQ):
                    out_stage[slot, :, pl.ds(q * CW, CW)] = (
                        in_stage[slot, q].astype(F32) * inv
                    )
                pltpu.make_async_copy(
                    out_stage.at[slot],
                    out_ref.at[:, pl.ds(out_col, GW)],
                    out_copy.at[slot],
                ).start()

            for slot in range(2):
                pltpu.make_async_copy(
                    out_stage.at[slot],
                    out_ref.at[:, pl.ds(slot * GW, GW)],
                    out_copy.at[slot],
                ).wait()
            for m in range(NM):
                z_rdma(m).wait_send()
                xd_out(m).wait_send()
                yd_out(m).wait_send()
            for m in range(4):
                yt_out(m).wait_send()
            for m in range(4, NM):
                xt_out(m).wait_send()
            stat_rdma().wait_send()

    return pl.pallas_call(
        body,
        grid=(NCHUNK,),
        in_specs=[
            pl.BlockSpec((T, D), lambda j: (0, 0)),
            pl.BlockSpec((D, CW), lambda j: (0, j)),
        ],
        out_specs=[
            pl.BlockSpec(memory_space=pl.ANY),
            pl.BlockSpec(memory_space=pl.ANY),
            pl.BlockSpec(memory_space=pl.ANY),
        ],
        out_shape=[
            jax.ShapeDtypeStruct((T, 2 * VH), F32),
            jax.ShapeDtypeStruct((NCHUNK, T, CW), jnp.bfloat16),
            jax.ShapeDtypeStruct((NCHUNK, T, CW), jnp.bfloat16),
        ],
        scratch_shapes=[
            pltpu.VMEM((NM, T, CW), jnp.bfloat16),
            pltpu.VMEM((2, T, CW), jnp.bfloat16),
            pltpu.VMEM((T, 1), F32),
            pltpu.VMEM((T, 1), F32),
            pltpu.VMEM((2, T, GW), F32),
            pltpu.VMEM((2, NQ, T, CW), jnp.bfloat16),
            pltpu.SemaphoreType.DMA((NM,)),
            pltpu.SemaphoreType.DMA((NM,)),
            pltpu.SemaphoreType.DMA((NM,)),
            pltpu.SemaphoreType.DMA((NM,)),
            pltpu.SemaphoreType.DMA((NM,)),
            pltpu.SemaphoreType.DMA((NM,)),
            pltpu.SemaphoreType.DMA((4,)),
            pltpu.SemaphoreType.DMA((4,)),
            pltpu.SemaphoreType.DMA((4,)),
            pltpu.SemaphoreType.DMA((4,)),
            pltpu.SemaphoreType.DMA,
            pltpu.SemaphoreType.DMA,
            pltpu.SemaphoreType.DMA((2,)),
            pltpu.SemaphoreType.DMA((2,)),
            pltpu.SemaphoreType.DMA((2,)),
        ],
        compiler_params=pltpu.CompilerParams(
            collective_id=0, vmem_limit_bytes=100 * 1024 * 1024
        ),
    )(x, W)


def kernel(x, W):
    out, _, _ = _fused(x, W)
    return out


# device time: 294959 ns/iter; 1.2332x vs baseline; 1.2332x over previous
import jax
import jax.numpy as jnp
from jax import lax
from jax.experimental import pallas as pl
from jax.experimental.pallas import tpu as pltpu

T = 1024
D = 2048
VH = 16384
NCHUNK = 16
CW = VH // NCHUNK
RB = 128


NQ = 4
NM = NCHUNK // NQ


def _gemm_send(x, W):

    def body(x_ref, w_ref, o_ref, recv_ref, comm_ref,
             z_send, z_recv, xd_send, xd_recv, yd_send, yd_recv,
             xt_send, xt_recv, yt_send, yt_recv):
        j = pl.program_id(0)
        my_x = lax.axis_index("x")
        my_y = lax.axis_index("y")
        my_z = lax.axis_index("z")
        zpeer = (my_x, my_y, 1 - my_z)
        xpeer = (1 - my_x, my_y, my_z)
        ypeer = (my_x, 1 - my_y, my_z)

        q_me = my_x + 2 * my_y
        q_xp = (1 - my_x) + 2 * my_y
        q_yp = my_x + 2 * (1 - my_y)
        q_d = (1 - my_x) + 2 * (1 - my_y)

        def cols(h):
            return recv_ref.at[:, pl.ds(h * CW, CW)]

        def rdma(src, h, send_sem, recv_sem, peer):
            return pltpu.make_async_remote_copy(
                src_ref=src,
                dst_ref=cols(h),
                send_sem=send_sem,
                recv_sem=recv_sem,
                device_id=peer,
                device_id_type=pl.DeviceIdType.MESH,
            )

        def z_rdma(m):
            return rdma(comm_ref.at[m], 4 * m + q_me,
                        z_send.at[m], z_recv.at[m], zpeer)

        def xd_out(m):
            h = 4 * m + q_me
            return rdma(cols(h), h, xd_send.at[m], xd_recv.at[m], xpeer)

        def yd_out(m):
            h = 4 * m + q_me
            return rdma(cols(h), h, yd_send.at[m], yd_recv.at[m], ypeer)

        def xd_in(m):
            h = 4 * m + q_xp
            return rdma(cols(h), h, xd_send.at[m], xd_recv.at[m], xpeer)

        def yd_in(m):
            h = 4 * m + q_yp
            return rdma(cols(h), h, yd_send.at[m], yd_recv.at[m], ypeer)

        def yt_out(m):
            h = 4 * m + q_xp
            return rdma(cols(h), h, yt_send.at[m], yt_recv.at[m], ypeer)

        def yt_in(m):
            h = 4 * m + q_d
            return rdma(cols(h), h, yt_send.at[m], yt_recv.at[m], ypeer)

        def xt_out(m):
            h = 4 * m + q_yp
            return rdma(cols(h), h, xt_send.at[m - 2], xt_recv.at[m - 2], xpeer)

        def xt_in(m):
            h = 4 * m + q_d
            return rdma(cols(h), h, xt_send.at[m - 2], xt_recv.at[m - 2], xpeer)

        @pl.when(j == 0)
        def _():
            barrier = pltpu.get_barrier_semaphore()
            for peer in (zpeer, xpeer, ypeer):
                pl.semaphore_signal(
                    barrier, inc=1, device_id=peer,
                    device_id_type=pl.DeviceIdType.MESH,
                )
            pl.semaphore_wait(barrier, 3)

        xb = x_ref[...].astype(jnp.bfloat16)
        wb = w_ref[...].astype(jnp.bfloat16)
        accb = jnp.dot(xb, wb, preferred_element_type=jnp.float32).astype(
            jnp.bfloat16
        )
        o_ref[...] = accb

        @pl.when(j % NQ == q_me)
        def _():
            m = j // NQ
            comm_ref[m] = accb
            z_rdma(m).start()

        @pl.when((j >= 6) & ((j - 6) % 3 == 0))
        def _():
            m = (j - 6) // 3
            z_rdma(m).wait_recv()
            xd_out(m).start()
            yd_out(m).start()

        @pl.when(j == NCHUNK - 1)
        def _():
            for m in (0, 1):
                xd_in(m).wait_recv()
                yt_out(m).start()
            for m in (2, 3):
                yd_in(m).wait_recv()
                xt_out(m).start()
            for m in (2, 3):
                xd_in(m).wait_recv()
            for m in (0, 1):
                yd_in(m).wait_recv()
            for m in (0, 1):
                yt_in(m).wait_recv()
            for m in (2, 3):
                xt_in(m).wait_recv()
            for m in range(NM):
                z_rdma(m).wait_send()
                xd_out(m).wait_send()
                yd_out(m).wait_send()
            for m in (0, 1):
                yt_out(m).wait_send()
            for m in (2, 3):
                xt_out(m).wait_send()

    return pl.pallas_call(
        body,
        grid=(NCHUNK,),
        in_specs=[
            pl.BlockSpec((T, D), lambda j: (0, 0)),
            pl.BlockSpec((D, CW), lambda j: (0, j)),
        ],
        out_specs=[
            pl.BlockSpec((T, CW), lambda j: (0, j)),
            pl.BlockSpec(memory_space=pl.ANY),
        ],
        out_shape=[
            jax.ShapeDtypeStruct((T, VH), jnp.bfloat16),
            jax.ShapeDtypeStruct((T, VH), jnp.bfloat16),
        ],
        scratch_shapes=[
            pltpu.VMEM((NM, T, CW), jnp.bfloat16),
            pltpu.SemaphoreType.DMA((NM,)),
            pltpu.SemaphoreType.DMA((NM,)),
            pltpu.SemaphoreType.DMA((NM,)),
            pltpu.SemaphoreType.DMA((NM,)),
            pltpu.SemaphoreType.DMA((NM,)),
            pltpu.SemaphoreType.DMA((NM,)),
            pltpu.SemaphoreType.DMA((2,)),
            pltpu.SemaphoreType.DMA((2,)),
            pltpu.SemaphoreType.DMA((2,)),
            pltpu.SemaphoreType.DMA((2,)),
        ],
        compiler_params=pltpu.CompilerParams(
            collective_id=0, vmem_limit_bytes=100 * 1024 * 1024
        ),
    )(x, W)


def _softmax(mine, other):

    def body(a_ref, b_ref, o_ref):
        my_z = lax.axis_index("z")
        a = a_ref[...].astype(jnp.float32)
        b = b_ref[...].astype(jnp.float32)
        m = jnp.maximum(
            jnp.max(a, axis=-1, keepdims=True),
            jnp.max(b, axis=-1, keepdims=True),
        )
        ea = jnp.exp(a - m)
        eb = jnp.exp(b - m)
        s = jnp.sum(ea, axis=-1, keepdims=True) + jnp.sum(eb, axis=-1, keepdims=True)
        pa = ea / s
        pb = eb / s

        @pl.when(my_z == 0)
        def _():
            o_ref[:, :VH] = pa
            o_ref[:, VH:] = pb

        @pl.when(my_z == 1)
        def _():
            o_ref[:, :VH] = pb
            o_ref[:, VH:] = pa

    return pl.pallas_call(
        body,
        grid=(T // RB,),
        in_specs=[
            pl.BlockSpec((RB, VH), lambda i: (i, 0)),
            pl.BlockSpec((RB, VH), lambda i: (i, 0)),
        ],
        out_specs=pl.BlockSpec((RB, 2 * VH), lambda i: (i, 0)),
        out_shape=jax.ShapeDtypeStruct((T, 2 * VH), jnp.float32),
        compiler_params=pltpu.CompilerParams(vmem_limit_bytes=100 * 1024 * 1024),
    )(mine, other)


def kernel(x, W):
    logits, other = _gemm_send(x, W)
    return _softmax(logits, other)
